# baseline (speedup 1.0000x reference)
import jax
import jax.numpy as jnp
from jax.experimental import pallas as pl
from jax.experimental.pallas import tpu as pltpu

O, P, G, N, H, D = 3, 128, 64, 20000, 512, 128
TEMP_INV = 10.0
EPS = 1e-12
QP = 4
GROUPS = P // QP
NSPLIT = 5
NCH = N // NSPLIT


def _proj_kernel(idx_ref, emb_ref, W1_ref, b1_ref, W2_ref, b2_ref,
                 z_ref, emb_vmem, acc3, sem):
    o = pl.program_id(0)
    for s in range(NSPLIT):
        pltpu.make_async_copy(
            emb_ref.at[o, pl.ds(s * NCH, NCH), :],
            emb_vmem.at[pl.ds(s * NCH, NCH), :],
            sem.at[s],
        ).start()
    for s in range(NSPLIT):
        pltpu.make_async_copy(
            emb_ref.at[o, pl.ds(s * NCH, NCH), :],
            emb_vmem.at[pl.ds(s * NCH, NCH), :],
            sem.at[s],
        ).wait()

    iota8 = jax.lax.broadcasted_iota(jnp.int32, (8, 1), 0)

    def group_body(pg, carry):
        base = o * (P * G) + pg * (QP * G)
        for q in range(QP):
            acc = jnp.zeros((8, H), jnp.float32)
            for g in range(G):
                i = idx_ref[base + q * G + g]
                ib = pl.multiple_of((i >> 3) << 3, 8)
                chunk = emb_vmem[pl.ds(ib, 8), :]
                acc = acc + jnp.where(iota8 == (i & 7), chunk, 0.0)
            acc3[pg * QP + q] = acc
        return carry

    jax.lax.fori_loop(0, GROUPS, group_body, 0)

    protos = jnp.sum(acc3[...], axis=1) * (1.0 / G)
    h = jnp.dot(protos, W1_ref[0], preferred_element_type=jnp.float32)
    h = jnp.maximum(h + b1_ref[0], 0.0)
    z = jnp.dot(h, W2_ref[0], preferred_element_type=jnp.float32) + b2_ref[0]
    z_ref[0] = z


def _loss_kernel(z_ref, out_ref):
    M = O * P
    z = z_ref[...].reshape(M, D)
    nrm = jnp.sqrt(jnp.sum(z * z, axis=1, keepdims=True))
    zn = z / (nrm + EPS)
    sim = jax.lax.dot_general(zn, zn, (((1,), (1,)), ((), ())),
                              preferred_element_type=jnp.float32) * TEMP_INV
    ri = jax.lax.broadcasted_iota(jnp.int32, (M, M), 0)
    ci = jax.lax.broadcasted_iota(jnp.int32, (M, M), 1)
    eye = ri == ci
    pos = ((ri % P) == (ci % P)) & (~eye)
    ex = jnp.where(eye, 0.0, jnp.exp(sim))
    denom = jnp.sum(ex, axis=1) + EPS
    possum = jnp.sum(jnp.where(pos, ex, 0.0), axis=1) + EPS
    out_ref[0, 0] = jnp.mean(jnp.log(denom) - jnp.log(possum))


@jax.jit
def kernel(emb, W1, b1, W2, b2, idx):
    idxf = idx.astype(jnp.int32).reshape(O * P * G)
    b1r = b1.reshape(O, 1, H)
    b2r = b2.reshape(O, 1, D)

    oo = jnp.arange(O, dtype=jnp.int32)[:, None, None]
    pp = jnp.arange(P, dtype=jnp.int32)[None, :, None]
    S = jnp.zeros((O, P, N), jnp.bfloat16).at[
        jnp.broadcast_to(oo, idx.shape),
        jnp.broadcast_to(pp, idx.shape),
        idx].add(jnp.bfloat16(1.0))
    probe = S.sum().astype(jnp.float32)

    z = pl.pallas_call(
        _proj_kernel,
        grid_spec=pltpu.PrefetchScalarGridSpec(
            num_scalar_prefetch=1,
            grid=(O,),
            in_specs=[
                pl.BlockSpec(memory_space=pltpu.MemorySpace.HBM),
                pl.BlockSpec((1, H, H), lambda o, c: (o, 0, 0)),
                pl.BlockSpec((1, 1, H), lambda o, c: (o, 0, 0)),
                pl.BlockSpec((1, H, D), lambda o, c: (o, 0, 0)),
                pl.BlockSpec((1, 1, D), lambda o, c: (o, 0, 0)),
            ],
            out_specs=pl.BlockSpec((1, P, D), lambda o, c: (o, 0, 0)),
            scratch_shapes=[
                pltpu.VMEM((N, H), jnp.float32),
                pltpu.VMEM((P, 8, H), jnp.float32),
                pltpu.SemaphoreType.DMA((NSPLIT,)),
            ],
        ),
        out_shape=jax.ShapeDtypeStruct((O, P, D), jnp.float32),
        compiler_params=pltpu.CompilerParams(
            dimension_semantics=("arbitrary",)),
    )(idxf, emb, W1, b1r, W2, b2r)

    loss = pl.pallas_call(
        _loss_kernel,
        in_specs=[pl.BlockSpec(memory_space=pltpu.MemorySpace.VMEM)],
        out_specs=pl.BlockSpec(memory_space=pltpu.MemorySpace.SMEM),
        out_shape=jax.ShapeDtypeStruct((1, 1), jnp.float32),
    )(z)
    return loss[0, 0] + 0.0 * probe


# speedup vs baseline: 3.0520x; 3.0520x over previous
import jax
import jax.numpy as jnp
from jax.experimental import pallas as pl
from jax.experimental.pallas import tpu as pltpu

O, P, G, N, H, D = 3, 128, 64, 20000, 512, 128
TEMP_INV = 10.0
EPS = 1e-12
QP = 4
GROUPS = P // QP
NSPLIT = 5
NCH = N // NSPLIT
M = O * P


def _fused_kernel(idx_ref, emb_ref, W1_ref, b1_ref, W2_ref, b2_ref,
                  out_ref, emb_vmem, acc3, zbuf, sem):
    o = pl.program_id(0)
    for s in range(NSPLIT):
        pltpu.make_async_copy(
            emb_ref.at[o, pl.ds(s * NCH, NCH), :],
            emb_vmem.at[pl.ds(s * NCH, NCH), :],
            sem.at[s],
        ).start()
    for s in range(NSPLIT):
        pltpu.make_async_copy(
            emb_ref.at[o, pl.ds(s * NCH, NCH), :],
            emb_vmem.at[pl.ds(s * NCH, NCH), :],
            sem.at[s],
        ).wait()

    iota8 = jax.lax.broadcasted_iota(jnp.int32, (8, 1), 0)

    def group_body(pg, carry):
        base = o * (P * G) + pg * (QP * G)
        for q in range(QP):
            acc = jnp.zeros((8, H), jnp.float32)
            for g in range(G):
                i = idx_ref[base + q * G + g]
                ib = pl.multiple_of((i >> 3) << 3, 8)
                chunk = emb_vmem[pl.ds(ib, 8), :]
                m = (iota8 == (i & 7)).astype(jnp.float32)
                acc = acc + chunk * m
            acc3[pg * QP + q] = acc
        return carry

    jax.lax.fori_loop(0, GROUPS, group_body, 0)

    protos = jnp.sum(acc3[...], axis=1) * (1.0 / G)
    h = jnp.dot(protos, W1_ref[0], preferred_element_type=jnp.float32)
    h = jnp.maximum(h + b1_ref[0], 0.0)
    z = jnp.dot(h, W2_ref[0], preferred_element_type=jnp.float32) + b2_ref[0]
    zbuf[pl.ds(o * P, P), :] = z

    @pl.when(o == O - 1)
    def _():
        zf = zbuf[...]
        nrm = jnp.sqrt(jnp.sum(zf * zf, axis=1, keepdims=True))
        zn = zf / (nrm + EPS)
        sim = jax.lax.dot_general(zn, zn, (((1,), (1,)), ((), ())),
                                  preferred_element_type=jnp.float32)
        sim = sim * TEMP_INV
        ri = jax.lax.broadcasted_iota(jnp.int32, (M, M), 0)
        ci = jax.lax.broadcasted_iota(jnp.int32, (M, M), 1)
        eye = ri == ci
        pos = ((ri % P) == (ci % P)) & (~eye)
        ex = jnp.where(eye, 0.0, jnp.exp(sim))
        denom = jnp.sum(ex, axis=1) + EPS
        possum = jnp.sum(jnp.where(pos, ex, 0.0), axis=1) + EPS
        out_ref[0, 0] = jnp.mean(jnp.log(denom) - jnp.log(possum))


@jax.jit
def kernel(emb, W1, b1, W2, b2, idx):
    idxf = idx.astype(jnp.int32).reshape(O * P * G)
    b1r = b1.reshape(O, 1, H)
    b2r = b2.reshape(O, 1, D)

    loss = pl.pallas_call(
        _fused_kernel,
        grid_spec=pltpu.PrefetchScalarGridSpec(
            num_scalar_prefetch=1,
            grid=(O,),
            in_specs=[
                pl.BlockSpec(memory_space=pltpu.MemorySpace.HBM),
                pl.BlockSpec((1, H, H), lambda o, c: (o, 0, 0)),
                pl.BlockSpec((1, 1, H), lambda o, c: (o, 0, 0)),
                pl.BlockSpec((1, H, D), lambda o, c: (o, 0, 0)),
                pl.BlockSpec((1, 1, D), lambda o, c: (o, 0, 0)),
            ],
            out_specs=pl.BlockSpec(memory_space=pltpu.MemorySpace.SMEM),
            scratch_shapes=[
                pltpu.VMEM((N, H), jnp.float32),
                pltpu.VMEM((P, 8, H), jnp.float32),
                pltpu.VMEM((M, D), jnp.float32),
                pltpu.SemaphoreType.DMA((NSPLIT,)),
            ],
        ),
        out_shape=jax.ShapeDtypeStruct((1, 1), jnp.float32),
        compiler_params=pltpu.CompilerParams(
            dimension_semantics=("arbitrary",)),
    )(idxf, emb, W1, b1r, W2, b2r)
    return loss[0, 0]


# speedup vs baseline: 3.0666x; 1.0048x over previous
import jax
import jax.numpy as jnp
from jax.experimental import pallas as pl
from jax.experimental.pallas import tpu as pltpu

O, P, G, N, H, D = 3, 128, 64, 20000, 512, 128
TEMP_INV = 10.0
EPS = 1e-12
QP = 8
GROUPS = P // QP
NSPLIT = 5
NCH = N // NSPLIT
M = O * P


def _fused_kernel(idx_ref, emb_ref, W1_ref, b1_ref, W2_ref, b2_ref,
                  out_ref, emb_vmem, acc3, zbuf, sem):
    o = pl.program_id(0)
    for s in range(NSPLIT):
        pltpu.make_async_copy(
            emb_ref.at[o, pl.ds(s * NCH, NCH), :],
            emb_vmem.at[pl.ds(s * NCH, NCH), :],
            sem.at[s],
        ).start()
    for s in range(NSPLIT):
        pltpu.make_async_copy(
            emb_ref.at[o, pl.ds(s * NCH, NCH), :],
            emb_vmem.at[pl.ds(s * NCH, NCH), :],
            sem.at[s],
        ).wait()

    iota8 = jax.lax.broadcasted_iota(jnp.int32, (8, 1), 0)

    def group_body(pg, carry):
        base = o * (P * G) + pg * (QP * G)
        for q in range(QP):
            acc = jnp.zeros((8, H), jnp.float32)
            for g in range(G):
                i = idx_ref[base + q * G + g]
                ib = pl.multiple_of((i >> 3) << 3, 8)
                chunk = emb_vmem[pl.ds(ib, 8), :]
                m = (iota8 == (i & 7)).astype(jnp.float32)
                acc = acc + chunk * m
            acc3[pg * QP + q] = acc
        return carry

    jax.lax.fori_loop(0, GROUPS, group_body, 0)

    protos = jnp.sum(acc3[...], axis=1) * (1.0 / G)
    h = jnp.dot(protos, W1_ref[0], preferred_element_type=jnp.float32)
    h = jnp.maximum(h + b1_ref[0], 0.0)
    z = jnp.dot(h, W2_ref[0], preferred_element_type=jnp.float32) + b2_ref[0]
    zbuf[pl.ds(o * P, P), :] = z

    @pl.when(o == O - 1)
    def _():
        zf = zbuf[...]
        nrm = jnp.sqrt(jnp.sum(zf * zf, axis=1, keepdims=True))
        zn = zf / (nrm + EPS)
        sim = jax.lax.dot_general(zn, zn, (((1,), (1,)), ((), ())),
                                  preferred_element_type=jnp.float32)
        sim = sim * TEMP_INV
        ri = jax.lax.broadcasted_iota(jnp.int32, (M, M), 0)
        ci = jax.lax.broadcasted_iota(jnp.int32, (M, M), 1)
        eye = ri == ci
        pos = ((ri % P) == (ci % P)) & (~eye)
        ex = jnp.where(eye, 0.0, jnp.exp(sim))
        denom = jnp.sum(ex, axis=1) + EPS
        possum = jnp.sum(jnp.where(pos, ex, 0.0), axis=1) + EPS
        out_ref[0, 0] = jnp.mean(jnp.log(denom) - jnp.log(possum))


@jax.jit
def kernel(emb, W1, b1, W2, b2, idx):
    idxf = idx.astype(jnp.int32).reshape(O * P * G)
    b1r = b1.reshape(O, 1, H)
    b2r = b2.reshape(O, 1, D)

    loss = pl.pallas_call(
        _fused_kernel,
        grid_spec=pltpu.PrefetchScalarGridSpec(
            num_scalar_prefetch=1,
            grid=(O,),
            in_specs=[
                pl.BlockSpec(memory_space=pltpu.MemorySpace.HBM),
                pl.BlockSpec((1, H, H), lambda o, c: (o, 0, 0)),
                pl.BlockSpec((1, 1, H), lambda o, c: (o, 0, 0)),
                pl.BlockSpec((1, H, D), lambda o, c: (o, 0, 0)),
                pl.BlockSpec((1, 1, D), lambda o, c: (o, 0, 0)),
            ],
            out_specs=pl.BlockSpec(memory_space=pltpu.MemorySpace.SMEM),
            scratch_shapes=[
                pltpu.VMEM((N, H), jnp.float32),
                pltpu.VMEM((P, 8, H), jnp.float32),
                pltpu.VMEM((M, D), jnp.float32),
                pltpu.SemaphoreType.DMA((NSPLIT,)),
            ],
        ),
        out_shape=jax.ShapeDtypeStruct((1, 1), jnp.float32),
        compiler_params=pltpu.CompilerParams(
            dimension_semantics=("arbitrary",)),
    )(idxf, emb, W1, b1r, W2, b2r)
    return loss[0, 0]


# speedup vs baseline: 3.1134x; 1.0153x over previous
import jax
import jax.numpy as jnp
from jax.experimental import pallas as pl
from jax.experimental.pallas import tpu as pltpu

O, P, G, N, H, D = 3, 128, 64, 20000, 512, 128
TEMP_INV = 10.0
EPS = 1e-12
QP = 8
GROUPS = P // QP
NSPLIT = 5
NCH = N // NSPLIT
M = O * P


def _fused_kernel(idx_ref, emb_ref, W1_ref, b1_ref, W2_ref, b2_ref,
                  out_ref, emb_vmem, acc3, zbuf, sem):
    o = pl.program_id(0)

    def _start(oj):
        for s in range(NSPLIT):
            pltpu.make_async_copy(
                emb_ref.at[oj, pl.ds(s * NCH, NCH), :],
                emb_vmem.at[pl.ds(s * NCH, NCH), :],
                sem.at[s],
            ).start()

    @pl.when(o == 0)
    def _():
        _start(o)

    for s in range(NSPLIT):
        pltpu.make_async_copy(
            emb_ref.at[o, pl.ds(s * NCH, NCH), :],
            emb_vmem.at[pl.ds(s * NCH, NCH), :],
            sem.at[s],
        ).wait()

    iota8 = jax.lax.broadcasted_iota(jnp.int32, (8, 1), 0)

    def group_body(pg, carry):
        base = o * (P * G) + pg * (QP * G)
        for q in range(QP):
            acc = jnp.zeros((8, H), jnp.float32)
            for g in range(G):
                i = idx_ref[base + q * G + g]
                ib = pl.multiple_of((i >> 3) << 3, 8)
                chunk = emb_vmem[pl.ds(ib, 8), :]
                m = (iota8 == (i & 7)).astype(jnp.float32)
                acc = acc + chunk * m
            acc3[pg * QP + q] = acc
        return carry

    jax.lax.fori_loop(0, GROUPS, group_body, 0)

    @pl.when(o < O - 1)
    def _():
        _start(o + 1)

    protos = jnp.sum(acc3[...], axis=1) * (1.0 / G)
    h = jnp.dot(protos, W1_ref[0], preferred_element_type=jnp.float32)
    h = jnp.maximum(h + b1_ref[0], 0.0)
    z = jnp.dot(h, W2_ref[0], preferred_element_type=jnp.float32) + b2_ref[0]
    zbuf[pl.ds(o * P, P), :] = z

    @pl.when(o == O - 1)
    def _():
        zf = zbuf[...]
        nrm = jnp.sqrt(jnp.sum(zf * zf, axis=1, keepdims=True))
        zn = zf / (nrm + EPS)
        sim = jax.lax.dot_general(zn, zn, (((1,), (1,)), ((), ())),
                                  preferred_element_type=jnp.float32)
        sim = sim * TEMP_INV
        ri = jax.lax.broadcasted_iota(jnp.int32, (M, M), 0)
        ci = jax.lax.broadcasted_iota(jnp.int32, (M, M), 1)
        eye = ri == ci
        pos = ((ri % P) == (ci % P)) & (~eye)
        ex = jnp.where(eye, 0.0, jnp.exp(sim))
        denom = jnp.sum(ex, axis=1) + EPS
        possum = jnp.sum(jnp.where(pos, ex, 0.0), axis=1) + EPS
        out_ref[0, 0] = jnp.mean(jnp.log(denom) - jnp.log(possum))


@jax.jit
def kernel(emb, W1, b1, W2, b2, idx):
    idxf = idx.astype(jnp.int32).reshape(O * P * G)
    b1r = b1.reshape(O, 1, H)
    b2r = b2.reshape(O, 1, D)

    loss = pl.pallas_call(
        _fused_kernel,
        grid_spec=pltpu.PrefetchScalarGridSpec(
            num_scalar_prefetch=1,
            grid=(O,),
            in_specs=[
                pl.BlockSpec(memory_space=pltpu.MemorySpace.HBM),
                pl.BlockSpec((1, H, H), lambda o, c: (o, 0, 0)),
                pl.BlockSpec((1, 1, H), lambda o, c: (o, 0, 0)),
                pl.BlockSpec((1, H, D), lambda o, c: (o, 0, 0)),
                pl.BlockSpec((1, 1, D), lambda o, c: (o, 0, 0)),
            ],
            out_specs=pl.BlockSpec(memory_space=pltpu.MemorySpace.SMEM),
            scratch_shapes=[
                pltpu.VMEM((N, H), jnp.float32),
                pltpu.VMEM((P, 8, H), jnp.float32),
                pltpu.VMEM((M, D), jnp.float32),
                pltpu.SemaphoreType.DMA((NSPLIT,)),
            ],
        ),
        out_shape=jax.ShapeDtypeStruct((1, 1), jnp.float32),
        compiler_params=pltpu.CompilerParams(
            dimension_semantics=("arbitrary",)),
    )(idxf, emb, W1, b1r, W2, b2r)
    return loss[0, 0]
